# baseline (speedup 1.0000x reference)
import functools

import jax
import jax.numpy as jnp
from jax import lax
from jax.experimental import pallas as pl
from jax.experimental.pallas import tpu as pltpu
from jax.experimental.pallas import tpu_sc as plsc

_B = 32768
_D = 1024
_C = 8
_NBUF = 8


def _sc_gather(idx_flat, table):
    info = plsc.get_sparse_core_info()
    nc, ns = info.num_cores, info.num_subcores
    nw = nc * ns
    b_per_w = _B // nw
    n_chunks = b_per_w // _C
    n_outer = n_chunks // _NBUF
    mesh = plsc.VectorSubcoreMesh(core_axis_name="c", subcore_axis_name="s")

    @functools.partial(
        pl.kernel,
        mesh=mesh,
        out_type=jax.ShapeDtypeStruct((_B, _D), jnp.float32),
        scratch_types=[
            pltpu.VMEM((b_per_w,), jnp.int32),
            pltpu.VMEM((_NBUF, _C, _D), jnp.float32),
        ]
        + [pltpu.SemaphoreType.DMA] * (2 * _NBUF),
    )
    def k(table_hbm, idx_hbm, out_hbm, idx_v, rows_v, *sems):
        gsem, ssem = sems[:_NBUF], sems[_NBUF:]
        sid = lax.axis_index("s")
        wid = sid * nc + lax.axis_index("c")
        base = wid * b_per_w
        pltpu.sync_copy(idx_hbm.at[pl.ds(base, b_per_w)], idx_v)

        def gd(b, g):
            return pltpu.make_async_copy(
                table_hbm.at[idx_v.at[pl.ds(g * _C, _C)]],
                rows_v.at[b],
                gsem[b],
            )

        def sd(b, g):
            return pltpu.make_async_copy(
                rows_v.at[b], out_hbm.at[pl.ds(base + g * _C, _C)], ssem[b]
            )

        for b in range(_NBUF):
            gd(b, b).start()

        def round_(i, carry):
            g0 = i * _NBUF
            for b in range(_NBUF):
                gd(b, g0 + b).wait()
                sd(b, g0 + b).start()
            for b in range(_NBUF):
                sd(b, g0 + b).wait()

                @pl.when(g0 + b + _NBUF < n_chunks)
                def _():
                    gd(b, g0 + b + _NBUF).start()

            return carry

        lax.fori_loop(0, n_outer, round_, 0)

    return k(table, idx_flat)


def kernel(position_ids, wpe):
    idx = position_ids.reshape(-1).astype(jnp.int32)
    out = _sc_gather(idx, wpe)
    return out.reshape(position_ids.shape + (wpe.shape[1],))
